# baseline (speedup 1.0000x reference)
import functools

import jax
import jax.numpy as jnp
from jax import lax
from jax.experimental import pallas as pl
from jax.experimental.pallas import tpu as pltpu

EPS = 1e-5
_VMEM_LIMIT = 56 * 1024 * 1024


def _round_up(x, m):
    return (x + m - 1) // m * m


def _col_mask(h_t, n_rows, tile, mask):
    if mask:
        col = pl.program_id(0) * tile + lax.broadcasted_iota(
            jnp.int32, (1, tile), 1)
        h_t = jnp.where(col < n_rows, h_t, 0.0)
    return h_t


def _stats_t(h_t, stat_ref):
    s = jnp.sum(h_t, axis=1, keepdims=True)
    ss = jnp.sum(h_t * h_t, axis=1, keepdims=True)
    stat_ref[...] = jnp.transpose(
        jnp.concatenate([s, ss], axis=1), (1, 0))[None]


def _fold_scale_shift(stats_ref, n_rows, width, g_ref, be_ref):
    s = jnp.sum(stats_ref[...], axis=0)
    mu = s[0:1, :width] * (1.0 / n_rows)
    var = jnp.maximum(s[1:2, :width] * (1.0 / n_rows) - mu * mu, 0.0)
    scale = g_ref[...] * lax.rsqrt(var + EPS)
    shift = be_ref[...] - mu * scale
    return scale, shift


def _fold_layer1(stats0_ref, g0_ref, be0_ref, w1_ref, b1_ref,
                 n_rows, hid0, f0):
    sc0, sh0 = _fold_scale_shift(stats0_ref, n_rows, hid0, g0_ref, be0_ref)
    w1t = jnp.transpose(w1_ref[...], (1, 0)) * sc0
    hid1 = w1t.shape[0]
    w1t = jnp.concatenate(
        [w1t, jnp.zeros((hid1, f0 - hid0), jnp.float32)], axis=1)
    b1c = jnp.transpose(
        jnp.dot(sh0, w1_ref[...], preferred_element_type=jnp.float32)
        + b1_ref[...], (1, 0))
    return w1t.astype(jnp.bfloat16), b1c


def _l0_kernel(v1_ref, v2_ref, w0a_ref, w0b_ref, b0c_ref,
               h0t_ref, stat_ref, *, n_rows, tile, mask):
    ht = lax.dot_general(w0a_ref[...], v1_ref[...], (((1,), (0,)), ((), ())),
                         preferred_element_type=jnp.float32)
    ht = ht + lax.dot_general(w0b_ref[...], v2_ref[...],
                              (((1,), (0,)), ((), ())),
                              preferred_element_type=jnp.float32)
    ht = jnp.maximum(ht + b0c_ref[...], 0.0)
    ht = _col_mask(ht, n_rows, tile, mask)
    h0t_ref[...] = ht.astype(h0t_ref.dtype)
    _stats_t(ht, stat_ref)


def _l1_stats_kernel(h0t_ref, stats0_ref, g0_ref, be0_ref, w1_ref, b1_ref,
                     stat_ref, *, n_rows, tile, mask, hid0, f0):
    w1t, b1c = _fold_layer1(stats0_ref, g0_ref, be0_ref, w1_ref, b1_ref,
                            n_rows, hid0, f0)
    ht = lax.dot_general(w1t, h0t_ref[...], (((1,), (0,)), ((), ())),
                         preferred_element_type=jnp.float32)
    ht = jnp.maximum(ht + b1c, 0.0)
    ht = _col_mask(ht, n_rows, tile, mask)
    _stats_t(ht, stat_ref)


def _out_kernel(h0t_ref, stats0_ref, g0_ref, be0_ref, w1_ref, b1_ref,
                stats1_ref, g1_ref, be1_ref, w2_ref, b2_ref,
                out_ref, *, n_rows, hid0, hid1, f0):
    w1t, b1c = _fold_layer1(stats0_ref, g0_ref, be0_ref, w1_ref, b1_ref,
                            n_rows, hid0, f0)
    sc1, sh1 = _fold_scale_shift(stats1_ref, n_rows, hid1, g1_ref, be1_ref)
    w2row = jnp.transpose(w2_ref[...], (1, 0)) * sc1
    b2s = jnp.dot(sh1, w2_ref[...],
                  preferred_element_type=jnp.float32) + b2_ref[...]
    ht = lax.dot_general(w1t, h0t_ref[...], (((1,), (0,)), ((), ())),
                         preferred_element_type=jnp.float32)
    ht = jnp.maximum(ht + b1c, 0.0)
    z = lax.dot_general(w2row, ht, (((1,), (0,)), ((), ())),
                        preferred_element_type=jnp.float32) + b2s
    out_ref[...] = 1.0 / (1.0 + jnp.exp(-z))


def kernel(v1, v2, w0, b0, g0, be0, w1, b1, g1, be1, w2, b2):
    n, node_dim = v1.shape
    hid0 = w0.shape[1]
    hid1 = w1.shape[1]
    out_dim = w2.shape[1]
    f0 = _round_up(hid0, 32)

    tile = min(65536, _round_up(n, 512))
    n_pad = _round_up(n, tile)
    grid_n = n_pad // tile
    mask = n_pad != n
    if mask:
        v1 = jnp.pad(v1, ((0, n_pad - n), (0, 0)))
        v2 = jnp.pad(v2, ((0, n_pad - n), (0, 0)))
    v1t = v1.T
    v2t = v2.T

    cp = pltpu.CompilerParams(dimension_semantics=("arbitrary",),
                              vmem_limit_bytes=_VMEM_LIMIT)

    def rep(arr):
        return pl.BlockSpec(arr.shape, lambda i: (0,) * arr.ndim)

    def stat_spec(width):
        return pl.BlockSpec((1, 2, width), lambda i: (i, 0, 0))

    def stat_shape(width):
        return jax.ShapeDtypeStruct((grid_n, 2, width), jnp.float32)

    w0a = jnp.pad(w0[:node_dim], ((0, 0), (0, f0 - hid0))).T
    w0b = jnp.pad(w0[node_dim:], ((0, 0), (0, f0 - hid0))).T
    b0c = jnp.pad(b0, ((0, 0), (0, f0 - hid0))).T

    h0t, stats0 = pl.pallas_call(
        functools.partial(_l0_kernel, n_rows=n, tile=tile, mask=mask),
        grid=(grid_n,),
        in_specs=[pl.BlockSpec((node_dim, tile), lambda i: (0, i)),
                  pl.BlockSpec((node_dim, tile), lambda i: (0, i)),
                  rep(w0a), rep(w0b), rep(b0c)],
        out_specs=(pl.BlockSpec((f0, tile), lambda i: (0, i)),
                   stat_spec(f0)),
        out_shape=(jax.ShapeDtypeStruct((f0, n_pad), jnp.bfloat16),
                   stat_shape(f0)),
        compiler_params=cp,
    )(v1t, v2t, w0a, w0b, b0c)

    stats1 = pl.pallas_call(
        functools.partial(_l1_stats_kernel, n_rows=n, tile=tile, mask=mask,
                          hid0=hid0, f0=f0),
        grid=(grid_n,),
        in_specs=[pl.BlockSpec((f0, tile), lambda i: (0, i)),
                  rep(stats0), rep(g0), rep(be0), rep(w1), rep(b1)],
        out_specs=stat_spec(hid1),
        out_shape=stat_shape(hid1),
        compiler_params=cp,
    )(h0t, stats0, g0, be0, w1, b1)

    out_t = pl.pallas_call(
        functools.partial(_out_kernel, n_rows=n, hid0=hid0, hid1=hid1,
                          f0=f0),
        grid=(grid_n,),
        in_specs=[pl.BlockSpec((f0, tile), lambda i: (0, i)),
                  rep(stats0), rep(g0), rep(be0), rep(w1), rep(b1),
                  rep(stats1), rep(g1), rep(be1), rep(w2), rep(b2)],
        out_specs=pl.BlockSpec((out_dim, tile), lambda i: (0, i)),
        out_shape=jax.ShapeDtypeStruct((out_dim, n_pad), jnp.float32),
        compiler_params=cp,
    )(h0t, stats0, g0, be0, w1, b1, stats1, g1, be1, w2, b2)

    return out_t.T[:n]
